# Initial kernel scaffold; baseline (speedup 1.0000x reference)
import jax
import jax.numpy as jnp
from jax.experimental import pallas as pl


def kernel(clip_img_emb, t, encoded_txt, clip_txt_emb, pos_emb, final_emb, ln_gamma, ln_beta):
    raise NotImplementedError("write your pallas kernel here")


# speedup vs baseline: 2.9813x; 2.9813x over previous
import functools

import jax
import jax.numpy as jnp
from jax.experimental import pallas as pl
from jax.experimental.pallas import tpu as pltpu

B = 64
D = 1024
MAX_SEQ = 256
L_TXT = MAX_SEQ - 4


def _body(t_ref, txt_ref, ctxt_ref, img_ref, pe_ref, fin_ref, g_ref, b_ref,
          out_ref):
    b = pl.program_id(0)
    pe = pe_ref[...]
    txt = txt_ref[0]

    tval = t_ref[b].astype(jnp.float32)
    k = jax.lax.broadcasted_iota(jnp.int32, (1, D), 1)
    idx = jnp.where(k < D // 2, k, k - D // 2).astype(jnp.float32)
    inv_freq = jnp.exp(idx * (-jnp.log(10000.0) / (D // 2)))
    ang = tval * inv_freq
    temb = jnp.where(k < D // 2, jnp.sin(ang), jnp.cos(ang))

    bot = jnp.concatenate(
        [ctxt_ref[0], temb, img_ref[0], fin_ref[...]], axis=0)

    x = jnp.concatenate([txt, bot], axis=0) + pe

    mean = jnp.mean(x, axis=1, keepdims=True)
    xc = x - mean
    var = jnp.mean(xc * xc, axis=1, keepdims=True)
    y = xc * jax.lax.rsqrt(var + 1e-5) * g_ref[...] + b_ref[...]
    out_ref[0] = y


@jax.jit
def kernel(clip_img_emb, t, encoded_txt, clip_txt_emb, pos_emb, final_emb,
           ln_gamma, ln_beta):
    grid = (B,)
    out = pl.pallas_call(
        _body,
        grid=grid,
        in_specs=[
            pl.BlockSpec(memory_space=pltpu.SMEM),
            pl.BlockSpec((1, L_TXT, D), lambda b: (b, 0, 0)),
            pl.BlockSpec((1, 1, D), lambda b: (b, 0, 0)),
            pl.BlockSpec((1, 1, D), lambda b: (b, 0, 0)),
            pl.BlockSpec((MAX_SEQ, D), lambda b: (0, 0)),
            pl.BlockSpec((1, D), lambda b: (0, 0)),
            pl.BlockSpec((1, D), lambda b: (0, 0)),
            pl.BlockSpec((1, D), lambda b: (0, 0)),
        ],
        out_specs=pl.BlockSpec((1, MAX_SEQ, D), lambda b: (b, 0, 0)),
        out_shape=jax.ShapeDtypeStruct((B, MAX_SEQ, D), jnp.float32),
    )(t, encoded_txt, clip_txt_emb[:, None, :], clip_img_emb[:, None, :],
      pos_emb, final_emb[None, :], ln_gamma[None, :], ln_beta[None, :])
    return out


# speedup vs baseline: 2.9877x; 1.0021x over previous
import functools

import jax
import jax.numpy as jnp
from jax.experimental import pallas as pl
from jax.experimental.pallas import tpu as pltpu

B = 64
D = 1024
MAX_SEQ = 256
L_TXT = MAX_SEQ - 4


def _body(t_ref, txt_ref, ctxt_ref, img_ref, pe_ref, fin_ref, g_ref, b_ref,
          out_ref):
    b = pl.program_id(0)
    pe = pe_ref[...]
    txt = txt_ref[0]

    tval = t_ref[b].astype(jnp.float32)
    k = jax.lax.broadcasted_iota(jnp.int32, (1, D), 1)
    idx = jnp.where(k < D // 2, k, k - D // 2).astype(jnp.float32)
    inv_freq = jnp.exp(idx * (-jnp.log(10000.0) / (D // 2)))
    ang = tval * inv_freq
    temb = jnp.where(k < D // 2, jnp.sin(ang), jnp.cos(ang))

    bot = jnp.concatenate(
        [ctxt_ref[0], temb, img_ref[0], fin_ref[...]], axis=0)

    x = jnp.concatenate([txt, bot], axis=0) + pe

    mean = jnp.mean(x, axis=1, keepdims=True)
    xc = x - mean
    var = jnp.mean(xc * xc, axis=1, keepdims=True)
    y = xc * jax.lax.rsqrt(var + 1e-5) * g_ref[...] + b_ref[...]
    out_ref[0] = y


@jax.jit
def kernel(clip_img_emb, t, encoded_txt, clip_txt_emb, pos_emb, final_emb,
           ln_gamma, ln_beta):
    grid = (B,)
    out = pl.pallas_call(
        _body,
        grid=grid,
        in_specs=[
            pl.BlockSpec(memory_space=pltpu.SMEM),
            pl.BlockSpec((1, L_TXT, D), lambda b: (b, 0, 0)),
            pl.BlockSpec((1, 1, D), lambda b: (b, 0, 0)),
            pl.BlockSpec((1, 1, D), lambda b: (b, 0, 0)),
            pl.BlockSpec((MAX_SEQ, D), lambda b: (0, 0)),
            pl.BlockSpec((1, D), lambda b: (0, 0)),
            pl.BlockSpec((1, D), lambda b: (0, 0)),
            pl.BlockSpec((1, D), lambda b: (0, 0)),
        ],
        out_specs=pl.BlockSpec((1, MAX_SEQ, D), lambda b: (b, 0, 0)),
        out_shape=jax.ShapeDtypeStruct((B, MAX_SEQ, D), jnp.float32),
        compiler_params=pltpu.CompilerParams(
            dimension_semantics=("parallel",)),
    )(t, encoded_txt, clip_txt_emb[:, None, :], clip_img_emb[:, None, :],
      pos_emb, final_emb[None, :], ln_gamma[None, :], ln_beta[None, :])
    return out


# speedup vs baseline: 3.6960x; 1.2371x over previous
import functools

import jax
import jax.numpy as jnp
from jax.experimental import pallas as pl
from jax.experimental.pallas import tpu as pltpu

B = 64
D = 1024
MAX_SEQ = 256
L_TXT = MAX_SEQ - 4


NB = 4


def _body(t_ref, txt_ref, ctxt_ref, img_ref, pe_ref, fin_ref, g_ref, b_ref,
          out_ref):
    bb = pl.program_id(0)
    pe = pe_ref[...]
    k = jax.lax.broadcasted_iota(jnp.int32, (1, D), 1)
    idx = jnp.where(k < D // 2, k, k - D // 2).astype(jnp.float32)
    inv_freq = jnp.exp(idx * (-jnp.log(10000.0) / (D // 2)))
    is_sin = k < D // 2

    for i in range(NB):
        txt = txt_ref[i]

        tval = t_ref[bb * NB + i].astype(jnp.float32)
        ang = tval * inv_freq
        temb = jnp.where(is_sin, jnp.sin(ang), jnp.cos(ang))

        bot = jnp.concatenate(
            [ctxt_ref[i], temb, img_ref[i], fin_ref[...]], axis=0)

        x = jnp.concatenate([txt, bot], axis=0) + pe

        mean = jnp.mean(x, axis=1, keepdims=True)
        xc = x - mean
        var = jnp.mean(xc * xc, axis=1, keepdims=True)
        y = xc * jax.lax.rsqrt(var + 1e-5) * g_ref[...] + b_ref[...]
        out_ref[i] = y


@jax.jit
def kernel(clip_img_emb, t, encoded_txt, clip_txt_emb, pos_emb, final_emb,
           ln_gamma, ln_beta):
    grid = (B // NB,)
    out = pl.pallas_call(
        _body,
        grid=grid,
        in_specs=[
            pl.BlockSpec(memory_space=pltpu.SMEM),
            pl.BlockSpec((NB, L_TXT, D), lambda b: (b, 0, 0)),
            pl.BlockSpec((NB, 1, D), lambda b: (b, 0, 0)),
            pl.BlockSpec((NB, 1, D), lambda b: (b, 0, 0)),
            pl.BlockSpec((MAX_SEQ, D), lambda b: (0, 0)),
            pl.BlockSpec((1, D), lambda b: (0, 0)),
            pl.BlockSpec((1, D), lambda b: (0, 0)),
            pl.BlockSpec((1, D), lambda b: (0, 0)),
        ],
        out_specs=pl.BlockSpec((NB, MAX_SEQ, D), lambda b: (b, 0, 0)),
        out_shape=jax.ShapeDtypeStruct((B, MAX_SEQ, D), jnp.float32),
        compiler_params=pltpu.CompilerParams(
            dimension_semantics=("parallel",)),
    )(t, encoded_txt, clip_txt_emb[:, None, :], clip_img_emb[:, None, :],
      pos_emb, final_emb[None, :], ln_gamma[None, :], ln_beta[None, :])
    return out


# speedup vs baseline: 3.7859x; 1.0243x over previous
import functools

import jax
import jax.numpy as jnp
from jax.experimental import pallas as pl
from jax.experimental.pallas import tpu as pltpu

B = 64
D = 1024
MAX_SEQ = 256
L_TXT = MAX_SEQ - 4


NB = 8


def _body(t_ref, txt_ref, ctxt_ref, img_ref, pe_ref, fin_ref, g_ref, b_ref,
          out_ref):
    bb = pl.program_id(0)
    pe = pe_ref[...]
    k = jax.lax.broadcasted_iota(jnp.int32, (1, D), 1)
    idx = jnp.where(k < D // 2, k, k - D // 2).astype(jnp.float32)
    inv_freq = jnp.exp(idx * (-jnp.log(10000.0) / (D // 2)))
    is_sin = k < D // 2

    for i in range(NB):
        txt = txt_ref[i]

        tval = t_ref[bb * NB + i].astype(jnp.float32)
        ang = tval * inv_freq
        temb = jnp.where(is_sin, jnp.sin(ang), jnp.cos(ang))

        bot = jnp.concatenate(
            [ctxt_ref[i], temb, img_ref[i], fin_ref[...]], axis=0)

        x = jnp.concatenate([txt, bot], axis=0) + pe

        mean = jnp.mean(x, axis=1, keepdims=True)
        xc = x - mean
        var = jnp.mean(xc * xc, axis=1, keepdims=True)
        y = xc * jax.lax.rsqrt(var + 1e-5) * g_ref[...] + b_ref[...]
        out_ref[i] = y


@jax.jit
def kernel(clip_img_emb, t, encoded_txt, clip_txt_emb, pos_emb, final_emb,
           ln_gamma, ln_beta):
    grid = (B // NB,)
    out = pl.pallas_call(
        _body,
        grid=grid,
        in_specs=[
            pl.BlockSpec(memory_space=pltpu.SMEM),
            pl.BlockSpec((NB, L_TXT, D), lambda b: (b, 0, 0)),
            pl.BlockSpec((NB, 1, D), lambda b: (b, 0, 0)),
            pl.BlockSpec((NB, 1, D), lambda b: (b, 0, 0)),
            pl.BlockSpec((MAX_SEQ, D), lambda b: (0, 0)),
            pl.BlockSpec((1, D), lambda b: (0, 0)),
            pl.BlockSpec((1, D), lambda b: (0, 0)),
            pl.BlockSpec((1, D), lambda b: (0, 0)),
        ],
        out_specs=pl.BlockSpec((NB, MAX_SEQ, D), lambda b: (b, 0, 0)),
        out_shape=jax.ShapeDtypeStruct((B, MAX_SEQ, D), jnp.float32),
        compiler_params=pltpu.CompilerParams(
            dimension_semantics=("parallel",)),
    )(t, encoded_txt, clip_txt_emb[:, None, :], clip_img_emb[:, None, :],
      pos_emb, final_emb[None, :], ln_gamma[None, :], ln_beta[None, :])
    return out
